# baseline (speedup 1.0000x reference)
import jax
import jax.numpy as jnp
from jax.experimental import pallas as pl
from jax.experimental.pallas import tpu as pltpu


def _split3(a):
    hi = a.astype(jnp.bfloat16).astype(jnp.float32)
    r = a - hi
    mid = r.astype(jnp.bfloat16).astype(jnp.float32)
    lo = r - mid
    return hi, mid, lo


def _chamfer_kernel(x_ref, y_ref, out_ref):
    bs = x_ref.shape[0]
    n = x_ref.shape[1]
    total = jnp.float32(0.0)
    for b in range(bs):
        x = x_ref[b]
        y = y_ref[b]
        xsq = jnp.sum(x * x, axis=1, keepdims=True)
        ysq = jnp.sum(y * y, axis=1, keepdims=True)
        ones = jnp.ones((n, 1), dtype=jnp.float32)
        xa = jnp.concatenate([x, ones, xsq], axis=1)
        ya = jnp.concatenate([-2.0 * y, ysq, ones], axis=1)
        xs = _split3(xa)
        ys3 = _split3(ya)
        acat = jnp.concatenate(
            [xs[i] for i in range(3) for _ in range(3)], axis=1)
        bcat = jnp.concatenate(
            [ys3[j] for _ in range(3) for j in range(3)], axis=1)
        d2 = jax.lax.dot_general(
            acat, bcat, (((1,), (1,)), ((), ())),
            preferred_element_type=jnp.float32,
        )
        min_x = jnp.min(d2, axis=1)
        min_y = jnp.min(d2, axis=0)
        min_x = jnp.maximum(min_x, 0.0)
        min_y = jnp.maximum(min_y, 0.0)
        total += jnp.sum(jnp.sqrt(1e-6 + min_x))
        total += jnp.sum(jnp.sqrt(1e-6 + min_y))
    out_ref[0, 0] = total


def kernel(x, y):
    bs, n, _ = x.shape
    total = pl.pallas_call(
        _chamfer_kernel,
        out_specs=pl.BlockSpec(memory_space=pltpu.SMEM),
        out_shape=jax.ShapeDtypeStruct((1, 1), jnp.float32),
    )(x, y)
    return total[0, 0] / (bs * n)


# speedup vs baseline: 1.2550x; 1.2550x over previous
import jax
import jax.numpy as jnp
from jax.experimental import pallas as pl
from jax.experimental.pallas import tpu as pltpu


def _split3(a):
    hi = a.astype(jnp.bfloat16).astype(jnp.float32)
    r = a - hi
    mid = r.astype(jnp.bfloat16).astype(jnp.float32)
    lo = r - mid
    return hi, mid, lo


def _chamfer_kernel(x_ref, y_ref, out_ref):
    b = pl.program_id(0)
    x = x_ref[0]
    y = y_ref[0]
    n = x.shape[0]
    xsq = jnp.sum(x * x, axis=1, keepdims=True)
    ysq = jnp.sum(y * y, axis=1, keepdims=True)
    ones = jnp.ones((n, 1), dtype=jnp.float32)
    xa = jnp.concatenate([x, ones, xsq], axis=1)
    ya = jnp.concatenate([-2.0 * y, ysq, ones], axis=1)
    xs = _split3(xa)
    ys3 = _split3(ya)
    acat = jnp.concatenate(
        [xs[i] for i in range(3) for _ in range(3)], axis=1)
    bcat = jnp.concatenate(
        [ys3[j] for _ in range(3) for j in range(3)], axis=1)
    d2 = jax.lax.dot_general(
        acat, bcat, (((1,), (1,)), ((), ())),
        preferred_element_type=jnp.float32,
    )
    min_x = jnp.min(d2, axis=1)
    min_y = jnp.min(d2, axis=0)
    min_x = jnp.maximum(min_x, 0.0)
    min_y = jnp.maximum(min_y, 0.0)
    partial = jnp.sum(jnp.sqrt(1e-6 + min_x)) + jnp.sum(jnp.sqrt(1e-6 + min_y))

    @pl.when(b == 0)
    def _():
        out_ref[0, 0] = 0.0

    out_ref[0, 0] += partial


def kernel(x, y):
    bs, n, _ = x.shape
    total = pl.pallas_call(
        _chamfer_kernel,
        grid=(bs,),
        in_specs=[
            pl.BlockSpec((1, n, 3), lambda b: (b, 0, 0)),
            pl.BlockSpec((1, n, 3), lambda b: (b, 0, 0)),
        ],
        out_specs=pl.BlockSpec(memory_space=pltpu.SMEM),
        out_shape=jax.ShapeDtypeStruct((1, 1), jnp.float32),
    )(x, y)
    return total[0, 0] / (bs * n)


# speedup vs baseline: 1.7014x; 1.3557x over previous
import jax
import jax.numpy as jnp
from jax.experimental import pallas as pl
from jax.experimental.pallas import tpu as pltpu


def _split3(a):
    hi = a.astype(jnp.bfloat16)
    r = a - hi.astype(jnp.float32)
    mid = r.astype(jnp.bfloat16)
    lo = (r - mid.astype(jnp.float32)).astype(jnp.bfloat16)
    return hi, mid, lo


def _chamfer_kernel(a_ref, b_ref, out_ref):
    b = pl.program_id(0)
    acat = a_ref[0]
    bcat = b_ref[0]
    d2 = jax.lax.dot_general(
        acat, bcat, (((1,), (1,)), ((), ())),
        preferred_element_type=jnp.float32,
    )
    min_x = jnp.min(d2, axis=1)
    min_y = jnp.min(d2, axis=0)
    min_x = jnp.maximum(min_x, 0.0)
    min_y = jnp.maximum(min_y, 0.0)
    partial = jnp.sum(jnp.sqrt(1e-6 + min_x)) + jnp.sum(jnp.sqrt(1e-6 + min_y))

    @pl.when(b == 0)
    def _():
        out_ref[0, 0] = 0.0

    out_ref[0, 0] += partial


def kernel(x, y):
    bs, n, _ = x.shape
    xsq = jnp.sum(x * x, axis=2, keepdims=True)
    ysq = jnp.sum(y * y, axis=2, keepdims=True)
    ones = jnp.ones_like(xsq)
    xa = jnp.concatenate([x, ones, xsq], axis=2)
    ya = jnp.concatenate([-2.0 * y, ysq, ones], axis=2)
    xs = _split3(xa)
    ys3 = _split3(ya)
    acat = jnp.concatenate([xs[i] for i in range(3) for _ in range(3)], axis=2)
    bcat = jnp.concatenate([ys3[j] for _ in range(3) for j in range(3)], axis=2)

    k = acat.shape[2]
    total = pl.pallas_call(
        _chamfer_kernel,
        grid=(bs,),
        in_specs=[
            pl.BlockSpec((1, n, k), lambda b: (b, 0, 0)),
            pl.BlockSpec((1, n, k), lambda b: (b, 0, 0)),
        ],
        out_specs=pl.BlockSpec(memory_space=pltpu.SMEM),
        out_shape=jax.ShapeDtypeStruct((1, 1), jnp.float32),
    )(acat, bcat)
    return total[0, 0] / (bs * n)
